# baseline (speedup 1.0000x reference)
import functools

import jax
import jax.numpy as jnp
from jax import lax
from jax.experimental import pallas as pl
from jax.experimental.pallas import tpu as pltpu
from jax.experimental.pallas import tpu_sc as plsc

ST = 8
NBUF = 4
GA = 104
GB = 96


def _lut_body(item_ref, pos_ref, gamma_ref, beta_ref, out_ref):
    st, d = item_ref.shape
    v = pos_ref.shape[0]
    emb = item_ref[...][:, None, :] + pos_ref[...][None, :, :]
    emb = emb.reshape(st * v, d)
    mean = jnp.mean(emb, axis=-1, keepdims=True)
    var = jnp.mean((emb - mean) ** 2, axis=-1, keepdims=True)
    h = (emb - mean) / jnp.sqrt(var + 1e-5)
    out_ref[...] = h * gamma_ref[...] + beta_ref[...]


def _build_lut(item_table, pos_table, gamma, beta, S, V, D):
    return pl.pallas_call(
        _lut_body,
        grid=(S // ST,),
        in_specs=[
            pl.BlockSpec((ST, D), lambda i: (i, 0)),
            pl.BlockSpec((V, D), lambda i: (0, 0)),
            pl.BlockSpec((1, D), lambda i: (0, 0)),
            pl.BlockSpec((1, D), lambda i: (0, 0)),
        ],
        out_specs=pl.BlockSpec((ST * V, D), lambda i: (i, 0)),
        out_shape=jax.ShapeDtypeStruct((S * V, D), jnp.float32),
    )(item_table, pos_table, gamma.reshape(1, D), beta.reshape(1, D))


def _make_sc_gather(B, S, V, D):
    info = plsc.get_sparse_core_info()
    nc, ns = info.num_cores, info.num_subcores
    nw = nc * ns
    bpw = B // nw
    n_outer = bpw // NBUF
    mesh = plsc.VectorSubcoreMesh(core_axis_name="c", subcore_axis_name="s")

    @functools.partial(
        pl.kernel,
        mesh=mesh,
        compiler_params=pltpu.CompilerParams(use_tc_tiling_on_sc=False),
        out_type=jax.ShapeDtypeStruct((B, S, D), jnp.float32),
        scratch_types=(
            [pltpu.VMEM((bpw * S,), jnp.int32)]
            + [pltpu.VMEM((112,), jnp.int32) for _ in range(NBUF)]
            + [pltpu.VMEM((GB,), jnp.int32) for _ in range(NBUF)]
            + [pltpu.VMEM((S, D), jnp.float32) for _ in range(NBUF)]
            + [pltpu.SemaphoreType.DMA for _ in range(2 * NBUF)]
        ),
    )
    def sc_gather(lut_hbm, xflat_hbm, out_hbm, xbuf, *rest):
        idxa_refs = rest[:NBUF]
        idxb_refs = rest[NBUF:2 * NBUF]
        row_refs = rest[2 * NBUF:3 * NBUF]
        gsems = rest[3 * NBUF:4 * NBUF]
        wsems = rest[4 * NBUF:5 * NBUF]

        wid = lax.axis_index("s") * nc + lax.axis_index("c")
        b0 = wid * bpw
        pltpu.sync_copy(xflat_hbm.at[pl.ds(b0 * S, bpw * S)], xbuf)

        iota = lax.iota(jnp.int32, 16)
        ivs = iota * V

        def fire_gather(t, j):
            off = t * S
            for k in range(7):
                xv = xbuf[pl.ds(off + k * 16, 16)]
                idxa_refs[j][pl.ds(k * 16, 16)] = xv + (k * 16 * V) + ivs
            for k in range(6):
                xv = xbuf[pl.ds(off + GA + k * 16, 16)]
                idxb_refs[j][pl.ds(k * 16, 16)] = xv + ((GA + k * 16) * V) + ivs
            pltpu.async_copy(
                lut_hbm.at[idxa_refs[j].at[pl.ds(0, GA)]],
                row_refs[j].at[pl.ds(0, GA)], gsems[j])
            pltpu.async_copy(
                lut_hbm.at[idxb_refs[j]],
                row_refs[j].at[pl.ds(GA, GB)], gsems[j])

        def fire_write(t, j):
            pltpu.async_copy(row_refs[j], out_hbm.at[b0 + t], wsems[j])

        def wait_gather(j):
            pltpu.make_async_copy(
                lut_hbm.at[idxa_refs[j].at[pl.ds(0, GA)]],
                row_refs[j].at[pl.ds(0, GA)], gsems[j]).wait()
            pltpu.make_async_copy(
                lut_hbm.at[idxb_refs[j]],
                row_refs[j].at[pl.ds(GA, GB)], gsems[j]).wait()

        def wait_write(j):
            pltpu.make_async_copy(row_refs[j], out_hbm.at[0], wsems[j]).wait()

        for j in range(NBUF):
            fire_gather(j, j)

        def outer(k, carry):
            t0 = k * NBUF
            for j in range(NBUF):
                wait_gather(j)
                fire_write(t0 + j, j)
            for j in range(NBUF):
                @pl.when(k < n_outer - 1)
                def _():
                    wait_write(j)
                    fire_gather(t0 + NBUF + j, j)
            return carry

        lax.fori_loop(0, n_outer, outer, 0)
        for j in range(NBUF):
            wait_write(j)

    return sc_gather


def kernel(x, item_table, pos_table, gamma, beta):
    B, S = x.shape
    V, D = pos_table.shape
    lut = _build_lut(item_table, pos_table, gamma, beta, S, V, D)
    sc_gather = _make_sc_gather(B, S, V, D)
    return sc_gather(lut, x.reshape(B * S))


# speedup vs baseline: 1.1145x; 1.1145x over previous
import functools

import jax
import jax.numpy as jnp
from jax import lax
from jax.experimental import pallas as pl
from jax.experimental.pallas import tpu as pltpu
from jax.experimental.pallas import tpu_sc as plsc

ST = 8
NBUF = 2
GA = 104
GB = 96
LW = 128


def _lut_body(item_ref, pos_ref, gamma_ref, beta_ref, out_ref):
    st, d = item_ref.shape
    v = pos_ref.shape[0]
    emb = item_ref[...][:, None, :] + pos_ref[...][None, :, :]
    emb = emb.reshape(st * v, d)
    mean = jnp.mean(emb, axis=-1, keepdims=True)
    var = jnp.mean((emb - mean) ** 2, axis=-1, keepdims=True)
    h = (emb - mean) / jnp.sqrt(var + 1e-5)
    res = h * gamma_ref[...] + beta_ref[...]
    out_ref[...] = jnp.concatenate([res, jnp.zeros_like(res)], axis=-1)


def _build_lut(item_table, pos_table, gamma, beta, S, V, D):
    return pl.pallas_call(
        _lut_body,
        grid=(S // ST,),
        in_specs=[
            pl.BlockSpec((ST, D), lambda i: (i, 0)),
            pl.BlockSpec((V, D), lambda i: (0, 0)),
            pl.BlockSpec((1, D), lambda i: (0, 0)),
            pl.BlockSpec((1, D), lambda i: (0, 0)),
        ],
        out_specs=pl.BlockSpec((ST * V, LW), lambda i: (i, 0)),
        out_shape=jax.ShapeDtypeStruct((S * V, LW), jnp.float32),
    )(item_table, pos_table, gamma.reshape(1, D), beta.reshape(1, D))


def _make_sc_gather(B, S, V, D):
    info = plsc.get_sparse_core_info()
    nc, ns = info.num_cores, info.num_subcores
    nw = nc * ns
    bpw = B // nw
    n_outer = bpw // NBUF
    mesh = plsc.VectorSubcoreMesh(core_axis_name="c", subcore_axis_name="s")

    @functools.partial(
        pl.kernel,
        mesh=mesh,
        out_type=jax.ShapeDtypeStruct((B, S, D), jnp.float32),
        scratch_types=(
            [pltpu.VMEM((208,), jnp.int32) for _ in range(NBUF)]
            + [pltpu.VMEM((112,), jnp.int32) for _ in range(NBUF)]
            + [pltpu.VMEM((GB,), jnp.int32) for _ in range(NBUF)]
            + [pltpu.VMEM((S, LW), jnp.float32) for _ in range(NBUF)]
            + [pltpu.VMEM((S, D), jnp.float32) for _ in range(NBUF)]
            + [pltpu.SemaphoreType.DMA for _ in range(3 * NBUF)]
        ),
    )
    def sc_gather(lut_hbm, xflat_hbm, out_hbm, *rest):
        xrow_refs = rest[:NBUF]
        idxa_refs = rest[NBUF:2 * NBUF]
        idxb_refs = rest[2 * NBUF:3 * NBUF]
        row_refs = rest[3 * NBUF:4 * NBUF]
        wbuf_refs = rest[4 * NBUF:5 * NBUF]
        xsems = rest[5 * NBUF:6 * NBUF]
        gsems = rest[6 * NBUF:7 * NBUF]
        wsems = rest[7 * NBUF:8 * NBUF]

        wid = lax.axis_index("s") * nc + lax.axis_index("c")
        b0 = wid * bpw

        iota = lax.iota(jnp.int32, 16)
        ivs = iota * V

        def fire_x(t, j):
            pltpu.async_copy(xflat_hbm.at[pl.ds((b0 + t) * S, S)],
                             xrow_refs[j].at[pl.ds(0, S)], xsems[j])

        def wait_x(j):
            pltpu.make_async_copy(xflat_hbm.at[pl.ds(0, S)],
                                  xrow_refs[j].at[pl.ds(0, S)],
                                  xsems[j]).wait()

        def fire_gather(j):
            for k in range(7):
                xv = xrow_refs[j][pl.ds(k * 16, 16)]
                idxa_refs[j][pl.ds(k * 16, 16)] = xv + (k * 16 * V) + ivs
            for k in range(6):
                xv = xrow_refs[j][pl.ds(GA + k * 16, 16)]
                idxb_refs[j][pl.ds(k * 16, 16)] = xv + ((GA + k * 16) * V) + ivs
            pltpu.async_copy(
                lut_hbm.at[idxa_refs[j].at[pl.ds(0, GA)]],
                row_refs[j].at[pl.ds(0, GA)], gsems[j])
            pltpu.async_copy(
                lut_hbm.at[idxb_refs[j]],
                row_refs[j].at[pl.ds(GA, GB)], gsems[j])

        def wait_gather(j):
            pltpu.make_async_copy(
                lut_hbm.at[idxa_refs[j].at[pl.ds(0, GA)]],
                row_refs[j].at[pl.ds(0, GA)], gsems[j]).wait()
            pltpu.make_async_copy(
                lut_hbm.at[idxb_refs[j]],
                row_refs[j].at[pl.ds(GA, GB)], gsems[j]).wait()

        def compact(j):
            def body(r, carry):
                for u in range(2):
                    for k in range(D // 16):
                        wbuf_refs[j][r + u, pl.ds(k * 16, 16)] = (
                            row_refs[j][r + u, pl.ds(k * 16, 16)])
                return carry
            lax.fori_loop(0, S // 2, lambda r, c: body(2 * r, c), 0)

        def fire_write(t, j):
            pltpu.async_copy(wbuf_refs[j], out_hbm.at[b0 + t], wsems[j])

        def wait_write(j):
            pltpu.make_async_copy(wbuf_refs[j], out_hbm.at[0],
                                  wsems[j]).wait()

        for j in range(NBUF):
            fire_x(j, j)

        def outer(k, carry):
            t0 = k * NBUF
            for j in range(NBUF):
                wait_x(j)
                fire_gather(j)
            for j in range(NBUF):
                wait_gather(j)

                @pl.when(k > 0)
                def _():
                    wait_write(j)
                compact(j)
                fire_write(t0 + j, j)

                @pl.when(k < n_outer - 1)
                def _():
                    fire_x(t0 + NBUF + j, j)
            return carry

        lax.fori_loop(0, n_outer, outer, 0)
        for j in range(NBUF):
            wait_write(j)

    return sc_gather


def kernel(x, item_table, pos_table, gamma, beta):
    B, S = x.shape
    V, D = pos_table.shape
    lut = _build_lut(item_table, pos_table, gamma, beta, S, V, D)
    sc_gather = _make_sc_gather(B, S, V, D)
    return sc_gather(lut, x.reshape(B * S))
